# baseline (speedup 1.0000x reference)
import functools

import jax
import jax.numpy as jnp
from jax import lax
from jax.experimental import pallas as pl
from jax.experimental.pallas import tpu as pltpu
from jax.experimental.pallas import tpu_sc as plsc

EMB = 64
LANES = 16
CHUNK = 128


def _make_sc_kernel(B, n_workers, n_cores):
    bpw = B // n_workers
    nch = bpw // CHUNK

    def body(uemb_hbm, iemb_hbm, uids_hbm, iids_hbm, out_hbm,
             uidx, iidx, urows, irows, outv, sem):
        wid = lax.axis_index("s") * n_cores + lax.axis_index("c")
        base = wid * bpw

        for c in range(nch):
            pltpu.sync_copy(uids_hbm.at[pl.ds(base + c * CHUNK, CHUNK)],
                            uidx.at[c])
            pltpu.sync_copy(iids_hbm.at[pl.ds(base + c * CHUNK, CHUNK)],
                            iidx.at[c])

        copies = []
        for c in range(nch):
            copies.append(
                pltpu.async_copy(uemb_hbm.at[uidx.at[c]],
                                 urows.at[pl.ds(c * CHUNK, CHUNK)], sem))
            copies.append(
                pltpu.async_copy(iemb_hbm.at[iidx.at[c]],
                                 irows.at[pl.ds(c * CHUNK, CHUNK)], sem))
        for cp in copies:
            cp.wait()

        lane = lax.iota(jnp.int32, LANES)

        def group_body(g, carry):
            out_vec = jnp.zeros((LANES,), jnp.float32)
            for r in range(LANES):
                row = g * LANES + r
                u0 = urows[row, pl.ds(0, LANES)]
                u1 = urows[row, pl.ds(LANES, LANES)]
                u2 = urows[row, pl.ds(2 * LANES, LANES)]
                u3 = urows[row, pl.ds(3 * LANES, LANES)]
                v0 = irows[row, pl.ds(0, LANES)]
                v1 = irows[row, pl.ds(LANES, LANES)]
                v2 = irows[row, pl.ds(2 * LANES, LANES)]
                v3 = irows[row, pl.ds(3 * LANES, LANES)]
                acc = (u0 * v0 + u1 * v1) + (u2 * v2 + u3 * v3)
                s = jnp.sum(acc)
                out_vec = jnp.where(lane == r, s, out_vec)
            out_vec = 1.0 / (1.0 + jnp.exp(-out_vec))
            outv[pl.ds(g * LANES, LANES)] = out_vec
            return carry

        lax.fori_loop(0, bpw // LANES, group_body, 0)

        pltpu.sync_copy(outv, out_hbm.at[pl.ds(base, bpw)])

    return body, bpw, nch


def kernel(user_emb, item_emb, user_ids, item_ids):
    B = user_ids.shape[0]
    info = plsc.get_sparse_core_info()
    n_cores, n_subcores = info.num_cores, info.num_subcores
    n_workers = n_cores * n_subcores

    body, bpw, nch = _make_sc_kernel(B, n_workers, n_cores)

    f = pl.kernel(
        body,
        out_type=jax.ShapeDtypeStruct((B,), jnp.float32),
        mesh=plsc.VectorSubcoreMesh(core_axis_name="c", subcore_axis_name="s"),
        compiler_params=pltpu.CompilerParams(
            needs_layout_passes=False, use_tc_tiling_on_sc=False),
        scratch_types=[
            pltpu.VMEM((nch, CHUNK), jnp.int32),
            pltpu.VMEM((nch, CHUNK), jnp.int32),
            pltpu.VMEM((bpw, EMB), jnp.float32),
            pltpu.VMEM((bpw, EMB), jnp.float32),
            pltpu.VMEM((bpw,), jnp.float32),
            pltpu.SemaphoreType.DMA,
        ],
    )
    return f(user_emb, item_emb,
             user_ids.astype(jnp.int32), item_ids.astype(jnp.int32))


# speedup vs baseline: 1.5664x; 1.5664x over previous
import jax
import jax.numpy as jnp
from jax import lax
from jax.experimental import pallas as pl
from jax.experimental.pallas import tpu as pltpu
from jax.experimental.pallas import tpu_sc as plsc

EMB = 64
LANES = 16


def _make_body(bpw, n_cores):
    n_groups = bpw // LANES

    def body(uemb, iemb, uids, iids, out_hbm,
             uidx, iidx, ubuf, vbuf, outv, sem_a, sem_b):
        wid = lax.axis_index("s") * n_cores + lax.axis_index("c")
        base = wid * bpw

        pltpu.sync_copy(uids.at[pl.ds(base, bpw)], uidx)
        pltpu.sync_copy(iids.at[pl.ds(base, bpw)], iidx)

        lane = lax.iota(jnp.int32, LANES)

        def fire(g, slot, sem):
            uv = uidx[pl.ds(g * LANES, LANES)]
            iv = iidx[pl.ds(g * LANES, LANES)]
            for k in range(LANES):
                pltpu.async_copy(uemb.at[uv[k]], ubuf.at[slot, k], sem)
                pltpu.async_copy(iemb.at[iv[k]], vbuf.at[slot, k], sem)

        def drain(slot, sem):
            pltpu.make_async_copy(uemb.at[pl.ds(0, LANES)], ubuf.at[slot],
                                  sem).wait()
            pltpu.make_async_copy(iemb.at[pl.ds(0, LANES)], vbuf.at[slot],
                                  sem).wait()

        def compute(g, slot):
            out_vec = jnp.zeros((LANES,), jnp.float32)
            for r in range(LANES):
                u0 = ubuf[slot, r, pl.ds(0, LANES)]
                u1 = ubuf[slot, r, pl.ds(LANES, LANES)]
                u2 = ubuf[slot, r, pl.ds(2 * LANES, LANES)]
                u3 = ubuf[slot, r, pl.ds(3 * LANES, LANES)]
                v0 = vbuf[slot, r, pl.ds(0, LANES)]
                v1 = vbuf[slot, r, pl.ds(LANES, LANES)]
                v2 = vbuf[slot, r, pl.ds(2 * LANES, LANES)]
                v3 = vbuf[slot, r, pl.ds(3 * LANES, LANES)]
                acc = (u0 * v0 + u1 * v1) + (u2 * v2 + u3 * v3)
                s = jnp.sum(acc)
                out_vec = jnp.where(lane == r, s, out_vec)
            out_vec = 1.0 / (1.0 + jnp.exp(-out_vec))
            outv[pl.ds(g * LANES, LANES)] = out_vec

        fire(0, 0, sem_a)

        def pair_body(t, carry):
            g0 = 2 * t
            g1 = g0 + 1
            fire(g1, 1, sem_b)
            drain(0, sem_a)
            compute(g0, 0)

            @pl.when(t < (n_groups // 2) - 1)
            def _():
                fire(g0 + 2, 0, sem_a)

            drain(1, sem_b)
            compute(g1, 1)
            return carry

        lax.fori_loop(0, n_groups // 2, pair_body, 0)

        pltpu.sync_copy(outv, out_hbm.at[pl.ds(base, bpw)])

    return body


def kernel(user_emb, item_emb, user_ids, item_ids):
    B = user_ids.shape[0]
    info = plsc.get_sparse_core_info()
    n_cores, n_subcores = info.num_cores, info.num_subcores
    bpw = B // (n_cores * n_subcores)

    f = pl.kernel(
        _make_body(bpw, n_cores),
        out_type=jax.ShapeDtypeStruct((B,), jnp.float32),
        mesh=plsc.VectorSubcoreMesh(core_axis_name="c", subcore_axis_name="s"),
        compiler_params=pltpu.CompilerParams(
            needs_layout_passes=False, use_tc_tiling_on_sc=True),
        scratch_types=[
            pltpu.VMEM((bpw,), jnp.int32),
            pltpu.VMEM((bpw,), jnp.int32),
            pltpu.VMEM((2, LANES, EMB), jnp.float32),
            pltpu.VMEM((2, LANES, EMB), jnp.float32),
            pltpu.VMEM((bpw,), jnp.float32),
            pltpu.SemaphoreType.DMA,
            pltpu.SemaphoreType.DMA,
        ],
    )
    return f(user_emb, item_emb,
             user_ids.astype(jnp.int32), item_ids.astype(jnp.int32))


# speedup vs baseline: 1.9616x; 1.2522x over previous
import jax
import jax.numpy as jnp
from jax import lax
from jax.experimental import pallas as pl
from jax.experimental.pallas import tpu as pltpu
from jax.experimental.pallas import tpu_sc as plsc

EMB = 64
LANES = 16
SLAB = 128


def _transpose_body(n_rows_padded, n_workers, n_cores):
    n_slabs = n_rows_padded // SLAB

    def body(src, out,
             in0, in1, out0, out1, isem0, isem1, osem0, osem1):
        wid = lax.axis_index("s") * n_cores + lax.axis_index("c")
        nk = (n_slabs - wid + n_workers - 1) // n_workers
        max_pairs = (n_slabs + n_workers - 1) // n_workers // 2 + 1

        jvecs = [j0 + lax.iota(jnp.int32, LANES) for j0 in range(0, EMB, LANES)]

        def fire_in(k, dst, sem):
            rt = wid + k * n_workers
            pltpu.async_copy(src.at[:, pl.ds(rt * SLAB, SLAB)], dst, sem)

        def wait_in(dst, sem):
            pltpu.make_async_copy(src.at[:, pl.ds(0, SLAB)], dst, sem).wait()

        def fire_out(k, sbuf, sem):
            rt = wid + k * n_workers
            pltpu.async_copy(sbuf, out.at[pl.ds(rt * SLAB, SLAB)], sem)

        def wait_out(sbuf, sem):
            pltpu.make_async_copy(sbuf, out.at[pl.ds(0, SLAB)], sem).wait()

        def transpose_slab(ibuf, obuf):
            def row_body(r, carry):
                for jvec in jvecs:
                    t = (jvec + r) & (SLAB - 1)
                    vals = plsc.load_gather(ibuf, [jvec, t])
                    plsc.store_scatter(obuf, [t, jvec], vals)
                return carry
            lax.fori_loop(0, SLAB, row_body, 0)

        fire_in(0, in0, isem0)
        fire_in(1, in1, isem1)

        def pair_body(p, carry):
            specs = ((2 * p, in0, out0, isem0, osem0),
                     (2 * p + 1, in1, out1, isem1, osem1))
            for k, ibuf, obuf, isem, osem in specs:
                @pl.when(k < nk)
                def _(k=k, ibuf=ibuf, obuf=obuf, isem=isem, osem=osem):
                    @pl.when(k >= 2)
                    def _():
                        wait_out(obuf, osem)
                    wait_in(ibuf, isem)
                    transpose_slab(ibuf, obuf)
                    fire_out(k, obuf, osem)

                    @pl.when(k + 2 < nk)
                    def _():
                        fire_in(k + 2, ibuf, isem)
            return carry

        lax.fori_loop(0, max_pairs, pair_body, 0)
        wait_out(out0, osem0)
        wait_out(out1, osem1)

    return body


def _gather_body(bpw, n_cores):
    n_groups = bpw // LANES

    def body(uemb, iemb, uids, iids, out_hbm,
             uidx, iidx, ubuf, vbuf, outv, sem_a, sem_b):
        wid = lax.axis_index("s") * n_cores + lax.axis_index("c")
        base = wid * bpw

        pltpu.sync_copy(uids.at[pl.ds(base, bpw)], uidx)
        pltpu.sync_copy(iids.at[pl.ds(base, bpw)], iidx)

        lane = lax.iota(jnp.int32, LANES)

        def fire(g, slot, sem):
            uv = uidx[pl.ds(g * LANES, LANES)]
            iv = iidx[pl.ds(g * LANES, LANES)]
            for k in range(LANES):
                pltpu.async_copy(uemb.at[uv[k]], ubuf.at[slot, k], sem)
                pltpu.async_copy(iemb.at[iv[k]], vbuf.at[slot, k], sem)

        def drain(slot, sem):
            pltpu.make_async_copy(uemb.at[pl.ds(0, LANES)], ubuf.at[slot],
                                  sem).wait()
            pltpu.make_async_copy(iemb.at[pl.ds(0, LANES)], vbuf.at[slot],
                                  sem).wait()

        def compute(g, slot):
            out_vec = jnp.zeros((LANES,), jnp.float32)
            for r in range(LANES):
                u0 = ubuf[slot, r, pl.ds(0, LANES)]
                u1 = ubuf[slot, r, pl.ds(LANES, LANES)]
                u2 = ubuf[slot, r, pl.ds(2 * LANES, LANES)]
                u3 = ubuf[slot, r, pl.ds(3 * LANES, LANES)]
                v0 = vbuf[slot, r, pl.ds(0, LANES)]
                v1 = vbuf[slot, r, pl.ds(LANES, LANES)]
                v2 = vbuf[slot, r, pl.ds(2 * LANES, LANES)]
                v3 = vbuf[slot, r, pl.ds(3 * LANES, LANES)]
                acc = (u0 * v0 + u1 * v1) + (u2 * v2 + u3 * v3)
                s = jnp.sum(acc)
                out_vec = jnp.where(lane == r, s, out_vec)
            out_vec = 1.0 / (1.0 + jnp.exp(-out_vec))
            outv[pl.ds(g * LANES, LANES)] = out_vec

        fire(0, 0, sem_a)

        def pair_body(t, carry):
            g0 = 2 * t
            g1 = g0 + 1
            fire(g1, 1, sem_b)
            drain(0, sem_a)
            compute(g0, 0)

            @pl.when(t < (n_groups // 2) - 1)
            def _():
                fire(g0 + 2, 0, sem_a)

            drain(1, sem_b)
            compute(g1, 1)
            return carry

        lax.fori_loop(0, n_groups // 2, pair_body, 0)

        pltpu.sync_copy(outv, out_hbm.at[pl.ds(base, bpw)])

    return body


def kernel(user_emb, item_emb, user_ids, item_ids):
    B = user_ids.shape[0]
    n_items = item_emb.shape[0]
    info = plsc.get_sparse_core_info()
    n_cores, n_subcores = info.num_cores, info.num_subcores
    n_workers = n_cores * n_subcores
    bpw = B // n_workers
    n_rows_padded = (n_items + SLAB - 1) // SLAB * SLAB

    mesh = plsc.VectorSubcoreMesh(core_axis_name="c", subcore_axis_name="s")
    params = pltpu.CompilerParams(
        needs_layout_passes=False, use_tc_tiling_on_sc=True,
        disable_bounds_checks=True)

    transpose = pl.kernel(
        _transpose_body(n_rows_padded, n_workers, n_cores),
        out_type=jax.ShapeDtypeStruct((n_rows_padded, EMB), jnp.float32),
        mesh=mesh,
        compiler_params=params,
        scratch_types=[
            pltpu.VMEM((EMB, SLAB), jnp.float32),
            pltpu.VMEM((EMB, SLAB), jnp.float32),
            pltpu.VMEM((SLAB, EMB), jnp.float32),
            pltpu.VMEM((SLAB, EMB), jnp.float32),
            pltpu.SemaphoreType.DMA,
            pltpu.SemaphoreType.DMA,
            pltpu.SemaphoreType.DMA,
            pltpu.SemaphoreType.DMA,
        ],
    )

    gather = pl.kernel(
        _gather_body(bpw, n_cores),
        out_type=jax.ShapeDtypeStruct((B,), jnp.float32),
        mesh=mesh,
        compiler_params=params,
        scratch_types=[
            pltpu.VMEM((bpw,), jnp.int32),
            pltpu.VMEM((bpw,), jnp.int32),
            pltpu.VMEM((2, LANES, EMB), jnp.float32),
            pltpu.VMEM((2, LANES, EMB), jnp.float32),
            pltpu.VMEM((bpw,), jnp.float32),
            pltpu.SemaphoreType.DMA,
            pltpu.SemaphoreType.DMA,
        ],
    )

    item_rm = transpose(item_emb.T)
    return gather(user_emb, item_rm,
                  user_ids.astype(jnp.int32), item_ids.astype(jnp.int32))
